# baseline (speedup 1.0000x reference)
import functools

import jax
import jax.numpy as jnp
from jax import lax
from jax.experimental import pallas as pl
from jax.experimental.pallas import tpu as pltpu
from jax.experimental.pallas import tpu_sc as plsc

N = 50000
H = 512
FD = 16
B = 512

NC = 2
NS = 16
NW = NC * NS

R = 80
NBLK = N // R
NG = 8
NCG = NW // NG
CW = H // NCG
MAX_I = -(-NBLK // NG)
PAD_I = MAX_I + (MAX_I % 2)


def _sc_segment_sum(x, ids_pad):
    mesh = plsc.VectorSubcoreMesh(core_axis_name="c", subcore_axis_name="s")

    @functools.partial(
        pl.kernel,
        out_type=jax.ShapeDtypeStruct((NG, B, H), jnp.float32),
        mesh=mesh,
        scratch_types=[
            pltpu.VMEM((R, CW), jnp.float32),
            pltpu.VMEM((R, CW), jnp.float32),
            pltpu.VMEM((PAD_I, R), jnp.int32),
            pltpu.VMEM((B, CW), jnp.float32),
            pltpu.SemaphoreType.DMA,
            pltpu.SemaphoreType.DMA,
        ],
    )
    def k(x_hbm, ids_hbm, out_hbm, buf_a, buf_b, idx, acc, sem_a, sem_b):
        c = lax.axis_index("c")
        s = lax.axis_index("s")
        wid = s * NC + c
        g = wid % NG
        cg = wid // NG
        col = cg * CW

        pltpu.sync_copy(ids_hbm.at[g], idx)

        zv = jnp.zeros((16,), jnp.float32)

        def zbody(r, _):
            for q in range(CW // 16):
                acc[r, pl.ds(q * 16, 16)] = zv
            return 0

        lax.fori_loop(0, B, zbody, 0)

        def start(i, buf, sem):
            blk = g + NG * i

            @pl.when(blk < NBLK)
            def _():
                base = blk * R
                pltpu.async_copy(
                    x_hbm.at[pl.ds(base, R), pl.ds(col, CW)], buf, sem)

        def wait(i, buf, sem):
            @pl.when(g + NG * i < NBLK)
            def _():
                pltpu.make_async_copy(
                    x_hbm.at[pl.ds(0, R), pl.ds(col, CW)], buf, sem).wait()

        def process(i, buf):
            @pl.when(g + NG * i < NBLK)
            def _():
                for gg in range(R // 16):
                    r0 = gg * 16
                    ids_v = idx[i, pl.ds(r0, 16)]
                    i0 = ids_v[0]
                    i15 = ids_v[15]

                    @pl.when(i0 == i15)
                    def _():
                        for q in range(CW // 16):
                            t = buf[r0, pl.ds(q * 16, 16)]
                            for j in range(1, 16):
                                t = t + buf[r0 + j, pl.ds(q * 16, 16)]
                            plsc.addupdate(acc.at[i0, pl.ds(q * 16, 16)], t)

                    @pl.when(i0 != i15)
                    def _():
                        for j in range(16):
                            seg = ids_v[j]
                            for q in range(CW // 16):
                                v = buf[r0 + j, pl.ds(q * 16, 16)]
                                plsc.addupdate(
                                    acc.at[seg, pl.ds(q * 16, 16)], v)

        start(0, buf_a, sem_a)
        start(1, buf_b, sem_b)

        def pair_body(i2, _):
            ia = 2 * i2
            ib = ia + 1
            wait(ia, buf_a, sem_a)
            process(ia, buf_a)
            start(ia + 2, buf_a, sem_a)
            wait(ib, buf_b, sem_b)
            process(ib, buf_b)
            start(ib + 2, buf_b, sem_b)
            return 0

        lax.fori_loop(0, PAD_I // 2, pair_body, 0)

        pltpu.sync_copy(acc, out_hbm.at[g, :, pl.ds(col, CW)])

    return k(x, ids_pad)


def _tc_film_heads(partials, domain_emb, gamma_W, gamma_b, beta_W, beta_b,
                   h0_W1, h0_b1, h0_W2, h0_b2, h0_W3, h0_b3,
                   h1_W1, h1_b1, h1_W2, h1_b2, h1_W3, h1_b3):
    def body(p_ref, de_ref, gw_ref, gb_ref, bw_ref, bb_ref,
             w1a_ref, b1a_ref, w2a_ref, b2a_ref, w3a_ref, b3a_ref,
             w1b_ref, b1b_ref, w2b_ref, b2b_ref, w3b_ref, b3b_ref,
             out_ref):
        f32 = jnp.float32
        dn = (((1,), (1,)), ((), ()))
        hi = lax.Precision.HIGHEST

        gf = p_ref[0]
        for gi in range(1, NG):
            gf = gf + p_ref[gi]
        de = de_ref[...]
        gamma = lax.dot_general(de, gw_ref[...], dn, precision=hi,
                                preferred_element_type=f32) + gb_ref[...]
        beta = lax.dot_general(de, bw_ref[...], dn, precision=hi,
                               preferred_element_type=f32) + bb_ref[...]
        gf = gamma * gf + beta

        def head(w1, b1, w2, b2, w3, b3):
            h1 = lax.dot_general(gf, w1[...], dn, precision=hi,
                                 preferred_element_type=f32) + b1[...]
            h1 = h1 * jax.nn.sigmoid(h1)
            h2 = lax.dot_general(h1, w2[...], dn, precision=hi,
                                 preferred_element_type=f32) + b2[...]
            h2 = h2 * jax.nn.sigmoid(h2)
            return jnp.sum(h2 * w3[...], axis=1) + b3[0, 0]

        out_ref[0, :] = head(w1a_ref, b1a_ref, w2a_ref, b2a_ref,
                             w3a_ref, b3a_ref)
        out_ref[1, :] = head(w1b_ref, b1b_ref, w2b_ref, b2b_ref,
                             w3b_ref, b3b_ref)

    args = (partials, domain_emb,
            gamma_W, gamma_b.reshape(1, H), beta_W, beta_b.reshape(1, H),
            h0_W1, h0_b1.reshape(1, H), h0_W2, h0_b2.reshape(1, H // 2),
            h0_W3, h0_b3.reshape(1, 1),
            h1_W1, h1_b1.reshape(1, H), h1_W2, h1_b2.reshape(1, H // 2),
            h1_W3, h1_b3.reshape(1, 1))

    return pl.pallas_call(
        body,
        out_shape=jax.ShapeDtypeStruct((2, B), jnp.float32),
    )(*args)


def kernel(x, batch, domain_emb, gamma_W, gamma_b, beta_W, beta_b,
           h0_W1, h0_b1, h0_W2, h0_b2, h0_W3, h0_b3,
           h1_W1, h1_b1, h1_W2, h1_b2, h1_W3, h1_b3):
    ids = batch.astype(jnp.int32)
    blocks = ids.reshape(NBLK, R)
    order = jnp.arange(NG)[:, None] + NG * jnp.arange(PAD_I)[None, :]
    ids_pad = blocks[jnp.minimum(order, NBLK - 1)]
    partials = _sc_segment_sum(x, ids_pad)
    out = _tc_film_heads(partials, domain_emb, gamma_W, gamma_b, beta_W,
                         beta_b, h0_W1, h0_b1, h0_W2, h0_b2, h0_W3, h0_b3,
                         h1_W1, h1_b1, h1_W2, h1_b2, h1_W3, h1_b3)
    return out[0], out[1]


# speedup vs baseline: 2.8678x; 2.8678x over previous
import functools

import jax
import jax.numpy as jnp
from jax import lax
from jax.experimental import pallas as pl
from jax.experimental.pallas import tpu as pltpu
from jax.experimental.pallas import tpu_sc as plsc

N = 50000
H = 512
FD = 16
B = 512

NC = 2
NS = 16
NW = NC * NS

R = 80
NBLK = N // R
NG = 8
NCG = NW // NG
CW = H // NCG
MAX_I = -(-NBLK // NG)
PAD_I = 80
NSLOT = 4


def _sc_segment_sum(x, ids_pad):
    mesh = plsc.VectorSubcoreMesh(core_axis_name="c", subcore_axis_name="s")

    @functools.partial(
        pl.kernel,
        out_type=jax.ShapeDtypeStruct((NG, B, H), jnp.float32),
        mesh=mesh,
        scratch_types=[
            pltpu.VMEM((NSLOT, R, CW), jnp.float32),
            pltpu.VMEM((PAD_I, R), jnp.int32),
            pltpu.VMEM((B, CW), jnp.float32),
            pltpu.SemaphoreType.DMA((NSLOT,)),
        ],
    )
    def k(x_hbm, ids_hbm, out_hbm, buf, idx, acc, sems):
        c = lax.axis_index("c")
        s = lax.axis_index("s")
        wid = s * NC + c
        g = wid % NG
        cg = wid // NG
        col = cg * CW

        def start(i):
            sl = lax.rem(i, NSLOT)
            blk = g + NG * i

            @pl.when(blk < NBLK)
            def _():
                base = blk * R
                pltpu.async_copy(
                    x_hbm.at[pl.ds(base, R), pl.ds(col, CW)],
                    buf.at[sl], sems.at[sl])

        for i in range(NSLOT):
            start(i)

        pltpu.sync_copy(ids_hbm.at[g], idx)

        zv = jnp.zeros((16,), jnp.float32)

        def zbody(r, _):
            for q in range(CW // 16):
                acc[r, pl.ds(q * 16, 16)] = zv
            return 0

        lax.fori_loop(0, B, zbody, 0)

        def body(i, _):
            sl = lax.rem(i, NSLOT)

            @pl.when(g + NG * i < NBLK)
            def _():
                pltpu.make_async_copy(
                    x_hbm.at[pl.ds(0, R), pl.ds(col, CW)],
                    buf.at[sl], sems.at[sl]).wait()

                def rbody(gg, _):
                    r0 = gg * 16
                    ids_v = idx[i, pl.ds(r0, 16)]
                    i0 = ids_v[0]
                    i15 = ids_v[15]

                    @pl.when(i0 == i15)
                    def _():
                        for q in range(CW // 16):
                            t = buf[sl, r0, pl.ds(q * 16, 16)]
                            for j in range(1, 16):
                                t = t + buf[sl, r0 + j, pl.ds(q * 16, 16)]
                            plsc.addupdate(acc.at[i0, pl.ds(q * 16, 16)], t)

                    @pl.when(i0 != i15)
                    def _():
                        for j in range(16):
                            seg = ids_v[j]
                            for q in range(CW // 16):
                                v = buf[sl, r0 + j, pl.ds(q * 16, 16)]
                                plsc.addupdate(
                                    acc.at[seg, pl.ds(q * 16, 16)], v)

                    return 0

                lax.fori_loop(0, R // 16, rbody, 0)

            start(i + NSLOT)
            return 0

        lax.fori_loop(0, PAD_I, body, 0)

        pltpu.sync_copy(acc, out_hbm.at[g, :, pl.ds(col, CW)])

    return k(x, ids_pad)


def _tc_film_heads(partials, domain_emb, gamma_W, gamma_b, beta_W, beta_b,
                   h0_W1, h0_b1, h0_W2, h0_b2, h0_W3, h0_b3,
                   h1_W1, h1_b1, h1_W2, h1_b2, h1_W3, h1_b3):
    def body(p_ref, de_ref, gw_ref, gb_ref, bw_ref, bb_ref,
             w1a_ref, b1a_ref, w2a_ref, b2a_ref, w3a_ref, b3a_ref,
             w1b_ref, b1b_ref, w2b_ref, b2b_ref, w3b_ref, b3b_ref,
             out_ref):
        f32 = jnp.float32
        dn = (((1,), (1,)), ((), ()))
        hi = lax.Precision.HIGHEST

        gf = p_ref[0]
        for gi in range(1, NG):
            gf = gf + p_ref[gi]
        de = de_ref[...]
        gamma = lax.dot_general(de, gw_ref[...], dn, precision=hi,
                                preferred_element_type=f32) + gb_ref[...]
        beta = lax.dot_general(de, bw_ref[...], dn, precision=hi,
                               preferred_element_type=f32) + bb_ref[...]
        gf = gamma * gf + beta

        def head(w1, b1, w2, b2, w3, b3):
            h1 = lax.dot_general(gf, w1[...], dn, precision=hi,
                                 preferred_element_type=f32) + b1[...]
            h1 = h1 * jax.nn.sigmoid(h1)
            h2 = lax.dot_general(h1, w2[...], dn, precision=hi,
                                 preferred_element_type=f32) + b2[...]
            h2 = h2 * jax.nn.sigmoid(h2)
            return jnp.sum(h2 * w3[...], axis=1) + b3[0, 0]

        out_ref[0, :] = head(w1a_ref, b1a_ref, w2a_ref, b2a_ref,
                             w3a_ref, b3a_ref)
        out_ref[1, :] = head(w1b_ref, b1b_ref, w2b_ref, b2b_ref,
                             w3b_ref, b3b_ref)

    args = (partials, domain_emb,
            gamma_W, gamma_b.reshape(1, H), beta_W, beta_b.reshape(1, H),
            h0_W1, h0_b1.reshape(1, H), h0_W2, h0_b2.reshape(1, H // 2),
            h0_W3, h0_b3.reshape(1, 1),
            h1_W1, h1_b1.reshape(1, H), h1_W2, h1_b2.reshape(1, H // 2),
            h1_W3, h1_b3.reshape(1, 1))

    return pl.pallas_call(
        body,
        out_shape=jax.ShapeDtypeStruct((2, B), jnp.float32),
    )(*args)


def kernel(x, batch, domain_emb, gamma_W, gamma_b, beta_W, beta_b,
           h0_W1, h0_b1, h0_W2, h0_b2, h0_W3, h0_b3,
           h1_W1, h1_b1, h1_W2, h1_b2, h1_W3, h1_b3):
    ids = batch.astype(jnp.int32)
    blocks = ids.reshape(NBLK, R)
    order = jnp.arange(NG)[:, None] + NG * jnp.arange(PAD_I)[None, :]
    ids_pad = blocks[jnp.minimum(order, NBLK - 1)]
    partials = _sc_segment_sum(x, ids_pad)
    out = _tc_film_heads(partials, domain_emb, gamma_W, gamma_b, beta_W,
                         beta_b, h0_W1, h0_b1, h0_W2, h0_b2, h0_W3, h0_b3,
                         h1_W1, h1_b1, h1_W2, h1_b2, h1_W3, h1_b3)
    return out[0], out[1]


# speedup vs baseline: 4.6513x; 1.6219x over previous
import functools

import jax
import jax.numpy as jnp
from jax import lax
from jax.experimental import pallas as pl
from jax.experimental.pallas import tpu as pltpu
from jax.experimental.pallas import tpu_sc as plsc

N = 50000
H = 512
FD = 16
B = 512

NC = 2
NS = 16
NW = NC * NS

R = 80
NBLK = N // R
NG = 8
NCG = NW // NG
CW = H // NCG
MAX_I = -(-NBLK // NG)
PAD_I = 80
NSLOT = 4


def _sc_segment_sum(x, ids_pad):
    mesh = plsc.VectorSubcoreMesh(core_axis_name="c", subcore_axis_name="s")

    @functools.partial(
        pl.kernel,
        out_type=jax.ShapeDtypeStruct((NG, B, H), jnp.float32),
        mesh=mesh,
        scratch_types=[
            pltpu.VMEM((NSLOT, R, CW), jnp.float32),
            pltpu.VMEM((PAD_I, R), jnp.int32),
            pltpu.VMEM((B, CW), jnp.float32),
            pltpu.SemaphoreType.DMA((NSLOT,)),
        ],
    )
    def k(x_hbm, ids_hbm, out_hbm, buf, idx, acc, sems):
        c = lax.axis_index("c")
        s = lax.axis_index("s")
        wid = s * NC + c
        g = wid % NG
        cg = wid // NG
        col = cg * CW

        def start(i):
            sl = lax.rem(i, NSLOT)
            blk = g + NG * i

            @pl.when(blk < NBLK)
            def _():
                base = blk * R
                pltpu.async_copy(
                    x_hbm.at[pl.ds(base, R), pl.ds(col, CW)],
                    buf.at[sl], sems.at[sl])

        for i in range(NSLOT):
            start(i)

        pltpu.sync_copy(ids_hbm.at[g], idx)

        zv = jnp.zeros((16,), jnp.float32)

        def zbody(r, _):
            for q in range(CW // 16):
                acc[r, pl.ds(q * 16, 16)] = zv
            return 0

        lax.fori_loop(0, B, zbody, 0)

        def body(i, _):
            sl = lax.rem(i, NSLOT)

            @pl.when(g + NG * i < NBLK)
            def _():
                pltpu.make_async_copy(
                    x_hbm.at[pl.ds(0, R), pl.ds(col, CW)],
                    buf.at[sl], sems.at[sl]).wait()

                @functools.partial(plsc.parallel_loop, 0, R // 16, unroll=2)
                def rbody(gg):
                    r0 = gg * 16
                    ids_v = idx[i, pl.ds(r0, 16)]
                    i0 = ids_v[0]
                    i15 = ids_v[15]

                    @pl.when(i0 == i15)
                    def _():
                        for q in range(CW // 16):
                            t = buf[sl, r0, pl.ds(q * 16, 16)]
                            for j in range(1, 16):
                                t = t + buf[sl, r0 + j, pl.ds(q * 16, 16)]
                            plsc.addupdate(acc.at[i0, pl.ds(q * 16, 16)], t)

                    @pl.when(i0 != i15)
                    def _():
                        for j in range(16):
                            seg = ids_v[j]
                            for q in range(CW // 16):
                                v = buf[sl, r0 + j, pl.ds(q * 16, 16)]
                                plsc.addupdate(
                                    acc.at[seg, pl.ds(q * 16, 16)], v)

            start(i + NSLOT)
            return 0

        lax.fori_loop(0, PAD_I, body, 0)

        pltpu.sync_copy(acc, out_hbm.at[g, :, pl.ds(col, CW)])

    return k(x, ids_pad)


def _tc_film_heads(partials, domain_emb, gamma_W, gamma_b, beta_W, beta_b,
                   h0_W1, h0_b1, h0_W2, h0_b2, h0_W3, h0_b3,
                   h1_W1, h1_b1, h1_W2, h1_b2, h1_W3, h1_b3):
    def body(p_ref, de_ref, gw_ref, gb_ref, bw_ref, bb_ref,
             w1a_ref, b1a_ref, w2a_ref, b2a_ref, w3a_ref, b3a_ref,
             w1b_ref, b1b_ref, w2b_ref, b2b_ref, w3b_ref, b3b_ref,
             out_ref):
        f32 = jnp.float32
        dn = (((1,), (1,)), ((), ()))
        hi = lax.Precision.HIGHEST

        gf = p_ref[0]
        for gi in range(1, NG):
            gf = gf + p_ref[gi]
        de = de_ref[...]
        gamma = lax.dot_general(de, gw_ref[...], dn, precision=hi,
                                preferred_element_type=f32) + gb_ref[...]
        beta = lax.dot_general(de, bw_ref[...], dn, precision=hi,
                               preferred_element_type=f32) + bb_ref[...]
        gf = gamma * gf + beta

        def head(w1, b1, w2, b2, w3, b3):
            h1 = lax.dot_general(gf, w1[...], dn, precision=hi,
                                 preferred_element_type=f32) + b1[...]
            h1 = h1 * jax.nn.sigmoid(h1)
            h2 = lax.dot_general(h1, w2[...], dn, precision=hi,
                                 preferred_element_type=f32) + b2[...]
            h2 = h2 * jax.nn.sigmoid(h2)
            return jnp.sum(h2 * w3[...], axis=1) + b3[0, 0]

        out_ref[0, :] = head(w1a_ref, b1a_ref, w2a_ref, b2a_ref,
                             w3a_ref, b3a_ref)
        out_ref[1, :] = head(w1b_ref, b1b_ref, w2b_ref, b2b_ref,
                             w3b_ref, b3b_ref)

    args = (partials, domain_emb,
            gamma_W, gamma_b.reshape(1, H), beta_W, beta_b.reshape(1, H),
            h0_W1, h0_b1.reshape(1, H), h0_W2, h0_b2.reshape(1, H // 2),
            h0_W3, h0_b3.reshape(1, 1),
            h1_W1, h1_b1.reshape(1, H), h1_W2, h1_b2.reshape(1, H // 2),
            h1_W3, h1_b3.reshape(1, 1))

    return pl.pallas_call(
        body,
        out_shape=jax.ShapeDtypeStruct((2, B), jnp.float32),
    )(*args)


def kernel(x, batch, domain_emb, gamma_W, gamma_b, beta_W, beta_b,
           h0_W1, h0_b1, h0_W2, h0_b2, h0_W3, h0_b3,
           h1_W1, h1_b1, h1_W2, h1_b2, h1_W3, h1_b3):
    ids = batch.astype(jnp.int32)
    blocks = ids.reshape(NBLK, R)
    order = jnp.arange(NG)[:, None] + NG * jnp.arange(PAD_I)[None, :]
    ids_pad = blocks[jnp.minimum(order, NBLK - 1)]
    partials = _sc_segment_sum(x, ids_pad)
    out = _tc_film_heads(partials, domain_emb, gamma_W, gamma_b, beta_W,
                         beta_b, h0_W1, h0_b1, h0_W2, h0_b2, h0_W3, h0_b3,
                         h1_W1, h1_b1, h1_W2, h1_b2, h1_W3, h1_b3)
    return out[0], out[1]
